# baseline (speedup 1.0000x reference)
import jax
import jax.numpy as jnp
from jax.experimental import pallas as pl

_E = 64
_TOPK = 8
_SCALING = 32.0 / _TOPK


def _topk_gate(s):
    masked = s
    m = None
    for _ in range(_TOPK):
        cur = jnp.max(masked, axis=-1, keepdims=True)
        if m is None:
            m = cur
        masked = jnp.where(masked >= cur, -jnp.inf, masked)
    sel = jnp.isneginf(masked)
    e = jnp.where(sel, jnp.exp(s - m), 0.0)
    return e / jnp.sum(e, axis=-1, keepdims=True)


def _fused_body(x_ref, wa_ref, wb_ref, wr1_ref, wr2_ref, out_ref):
    x = x_ref[...]
    s = jnp.dot(
        jnp.dot(x, wr1_ref[...], preferred_element_type=jnp.float32),
        wr2_ref[...],
        preferred_element_type=jnp.float32,
    )
    gate = _topk_gate(s)
    z = jnp.dot(x, wa_ref[...], preferred_element_type=jnp.float32)
    out_ref[...] = (
        jnp.dot(z * gate, wb_ref[...], preferred_element_type=jnp.float32)
        * _SCALING
    )


def kernel(x, W_A, W_B, W_r1, W_r2):
    T, D = x.shape
    ER = W_A.shape[1]
    RD = W_r1.shape[1]
    BT = 1024
    grid = (T // BT,)
    return pl.pallas_call(
        _fused_body,
        grid=grid,
        in_specs=[
            pl.BlockSpec((BT, D), lambda i: (i, 0)),
            pl.BlockSpec((D, ER), lambda i: (0, 0)),
            pl.BlockSpec((ER, D), lambda i: (0, 0)),
            pl.BlockSpec((D, RD), lambda i: (0, 0)),
            pl.BlockSpec((RD, ER), lambda i: (0, 0)),
        ],
        out_specs=pl.BlockSpec((BT, D), lambda i: (i, 0)),
        out_shape=jax.ShapeDtypeStruct((T, D), x.dtype),
    )(x, W_A, W_B, W_r1, W_r2)


# speedup vs baseline: 1.1045x; 1.1045x over previous
import jax
import jax.numpy as jnp
from jax.experimental import pallas as pl

_E = 64
_TOPK = 8
_SCALING = 32.0 / _TOPK


def _topk_gate(s):
    masked = s
    m = None
    for _ in range(_TOPK - 1):
        cur = jnp.max(masked, axis=-1, keepdims=True)
        if m is None:
            m = cur
        masked = jnp.where(masked >= cur, -jnp.inf, masked)
    t8 = jnp.max(masked, axis=-1, keepdims=True)
    e = jnp.where(s >= t8, jnp.exp(s - m), 0.0)
    return e / jnp.sum(e, axis=-1, keepdims=True)


def _fused_body(x_ref, wa_ref, wb_ref, wr1_ref, wr2_ref, out_ref):
    x = x_ref[...]
    s = jnp.dot(
        jnp.dot(x, wr1_ref[...], preferred_element_type=jnp.float32),
        wr2_ref[...],
        preferred_element_type=jnp.float32,
    )
    gate = _topk_gate(s)
    z = jnp.dot(x, wa_ref[...], preferred_element_type=jnp.float32)
    out_ref[...] = (
        jnp.dot(z * gate, wb_ref[...], preferred_element_type=jnp.float32)
        * _SCALING
    )


def kernel(x, W_A, W_B, W_r1, W_r2):
    T, D = x.shape
    ER = W_A.shape[1]
    RD = W_r1.shape[1]
    BT = 2048
    grid = (T // BT,)
    return pl.pallas_call(
        _fused_body,
        grid=grid,
        in_specs=[
            pl.BlockSpec((BT, D), lambda i: (i, 0)),
            pl.BlockSpec((D, ER), lambda i: (0, 0)),
            pl.BlockSpec((ER, D), lambda i: (0, 0)),
            pl.BlockSpec((D, RD), lambda i: (0, 0)),
            pl.BlockSpec((RD, ER), lambda i: (0, 0)),
        ],
        out_specs=pl.BlockSpec((BT, D), lambda i: (i, 0)),
        out_shape=jax.ShapeDtypeStruct((T, D), x.dtype),
    )(x, W_A, W_B, W_r1, W_r2)


# speedup vs baseline: 1.1067x; 1.0019x over previous
import jax
import jax.numpy as jnp
from jax.experimental import pallas as pl
from jax.experimental.pallas import tpu as pltpu

_E = 64
_TOPK = 8
_SCALING = 32.0 / _TOPK


def _topk_gate(s):
    masked = s
    m = None
    for _ in range(_TOPK - 1):
        cur = jnp.max(masked, axis=-1, keepdims=True)
        if m is None:
            m = cur
        masked = jnp.where(masked >= cur, -jnp.inf, masked)
    t8 = jnp.max(masked, axis=-1, keepdims=True)
    e = jnp.where(s >= t8, jnp.exp(s - m), 0.0)
    return e / jnp.sum(e, axis=-1, keepdims=True)


def _fused_body(x_ref, wa_ref, wb_ref, wr1_ref, wr2_ref, out_ref):
    x = x_ref[...]
    s = jnp.dot(
        jnp.dot(x, wr1_ref[...], preferred_element_type=jnp.float32),
        wr2_ref[...],
        preferred_element_type=jnp.float32,
    )
    gate = _topk_gate(s)
    z = jnp.dot(x, wa_ref[...], preferred_element_type=jnp.float32)
    out_ref[...] = (
        jnp.dot(z * gate, wb_ref[...], preferred_element_type=jnp.float32)
        * _SCALING
    )


def kernel(x, W_A, W_B, W_r1, W_r2):
    T, D = x.shape
    ER = W_A.shape[1]
    RD = W_r1.shape[1]
    BT = 2048
    grid = (T // BT,)
    return pl.pallas_call(
        _fused_body,
        grid=grid,
        in_specs=[
            pl.BlockSpec((BT, D), lambda i: (i, 0)),
            pl.BlockSpec((D, ER), lambda i: (0, 0)),
            pl.BlockSpec((ER, D), lambda i: (0, 0)),
            pl.BlockSpec((D, RD), lambda i: (0, 0)),
            pl.BlockSpec((RD, ER), lambda i: (0, 0)),
        ],
        out_specs=pl.BlockSpec((BT, D), lambda i: (i, 0)),
        out_shape=jax.ShapeDtypeStruct((T, D), x.dtype),
        compiler_params=pltpu.CompilerParams(
            dimension_semantics=("parallel",),
        ),
    )(x, W_A, W_B, W_r1, W_r2)
